# baseline (speedup 1.0000x reference)
import functools

import jax
import jax.numpy as jnp
from jax import lax
from jax.experimental import pallas as pl
from jax.experimental.pallas import tpu as pltpu
from jax.experimental.pallas import tpu_sc as plsc

NUM_ENTITIES = 1000000
NUM_RELATIONS = 1000
EMBED_DIM = 64
MARGIN = 9.0
EPSILON = 2.0
BATCH = 16384
EMB_RANGE = (MARGIN + EPSILON) / EMBED_DIM
PI = 3.141592653589793

L = 16
NC, NS = 2, 16
NW = NC * NS
CPW = BATCH // NW
K = 64
NCHUNK = CPW // K
NDV = EMBED_DIM // L


def _trig_body(rel_ref, out_ref):
    phase = rel_ref[...] * (PI / EMB_RANGE)
    out_ref[...] = jnp.concatenate([jnp.cos(phase), jnp.sin(phase)], axis=-1)


def _trig_table(rel):
    return pl.pallas_call(
        _trig_body,
        out_shape=jax.ShapeDtypeStruct((NUM_RELATIONS, 2 * EMBED_DIM), jnp.float32),
    )(rel)


def _sqrt16(x):
    i = lax.bitcast_convert_type(x, jnp.int32)
    y = lax.bitcast_convert_type(
        jnp.int32(0x5F1FFFF9) - lax.shift_right_logical(i, 1), jnp.float32)
    y = (0.703952253 * y) * (2.38924456 - x * y * y)
    return x * y


def _sc_body(ent_hbm, trig_hbm, h_hbm, r_hbm, t_hbm, nt_hbm, pos_hbm, neg_hbm,
             hidx, ridx, tidx, ntidx,
             hrow0, trow0, tlrow0, ntrow0,
             hrow1, trow1, tlrow1, ntrow1,
             sump, sumn, posv, negv, sem0, sem1):
    wid = lax.axis_index("s") * NC + lax.axis_index("c")
    base = wid * CPW
    pltpu.sync_copy(h_hbm.at[pl.ds(base, CPW)], hidx)
    pltpu.sync_copy(r_hbm.at[pl.ds(base, CPW)], ridx)
    pltpu.sync_copy(t_hbm.at[pl.ds(base, CPW)], tidx)
    pltpu.sync_copy(nt_hbm.at[pl.ds(base, CPW)], ntidx)

    lane = lax.iota(jnp.int32, L)
    sets = ((hrow0, trow0, tlrow0, ntrow0), (hrow1, trow1, tlrow1, ntrow1))
    sems = (sem0, sem1)

    def issue(c):
        off = c * K
        hrow, trow, tlrow, ntrow = sets[c % 2]
        sem = sems[c % 2]
        return (
            pltpu.async_copy(ent_hbm.at[hidx.at[pl.ds(off, K)]], hrow, sem),
            pltpu.async_copy(trig_hbm.at[ridx.at[pl.ds(off, K)]], trow, sem),
            pltpu.async_copy(ent_hbm.at[tidx.at[pl.ds(off, K)]], tlrow, sem),
            pltpu.async_copy(ent_hbm.at[ntidx.at[pl.ds(off, K)]], ntrow, sem),
        )

    pending = {0: issue(0)}
    for c in range(NCHUNK):
        off = c * K
        hrow, trow, tlrow, ntrow = sets[c % 2]
        if c + 1 < NCHUNK:
            pending[c + 1] = issue(c + 1)
        for cp in pending.pop(c):
            cp.wait()

        @plsc.parallel_loop(0, K, unroll=3)
        def body(i, hrow=hrow, trow=trow, tlrow=tlrow, ntrow=ntrow):
            accp = jnp.zeros((L,), jnp.float32)
            accn = jnp.zeros((L,), jnp.float32)
            for d in range(NDV):
                s = d * L
                re_h = hrow[i, pl.ds(s, L)]
                im_h = hrow[i, pl.ds(EMBED_DIM + s, L)]
                cosr = trow[i, pl.ds(s, L)]
                sinr = trow[i, pl.ds(EMBED_DIM + s, L)]
                rot_re = re_h * cosr - im_h * sinr
                rot_im = re_h * sinr + im_h * cosr
                rep = rot_re - tlrow[i, pl.ds(s, L)]
                imp = rot_im - tlrow[i, pl.ds(EMBED_DIM + s, L)]
                accp = accp + _sqrt16(rep * rep + imp * imp)
                ren = rot_re - ntrow[i, pl.ds(s, L)]
                imn = rot_im - ntrow[i, pl.ds(EMBED_DIM + s, L)]
                accn = accn + _sqrt16(ren * ren + imn * imn)
            sump[pl.ds((off + i) * L, L)] = accp
            sumn[pl.ds((off + i) * L, L)] = accn

    @plsc.parallel_loop(0, CPW // L)
    def red(g):
        rowbase = (g * L + lane) * L
        totp = jnp.zeros((L,), jnp.float32)
        totn = jnp.zeros((L,), jnp.float32)
        for col in range(L):
            idx = rowbase + col
            totp = totp + plsc.load_gather(sump, [idx])
            totn = totn + plsc.load_gather(sumn, [idx])
        posv[pl.ds(g * L, L)] = MARGIN - totp
        negv[pl.ds(g * L, L)] = MARGIN - totn

    pltpu.sync_copy(posv, pos_hbm.at[pl.ds(base, CPW)])
    pltpu.sync_copy(negv, neg_hbm.at[pl.ds(base, CPW)])


@functools.partial(
    pl.kernel,
    out_type=(jax.ShapeDtypeStruct((BATCH,), jnp.float32),
              jax.ShapeDtypeStruct((BATCH,), jnp.float32)),
    mesh=plsc.VectorSubcoreMesh(core_axis_name="c", subcore_axis_name="s"),
    compiler_params=pltpu.CompilerParams(needs_layout_passes=False),
    scratch_types=[
        pltpu.VMEM((CPW,), jnp.int32),
        pltpu.VMEM((CPW,), jnp.int32),
        pltpu.VMEM((CPW,), jnp.int32),
        pltpu.VMEM((CPW,), jnp.int32),
        pltpu.VMEM((K, 2 * EMBED_DIM), jnp.float32),
        pltpu.VMEM((K, 2 * EMBED_DIM), jnp.float32),
        pltpu.VMEM((K, 2 * EMBED_DIM), jnp.float32),
        pltpu.VMEM((K, 2 * EMBED_DIM), jnp.float32),
        pltpu.VMEM((K, 2 * EMBED_DIM), jnp.float32),
        pltpu.VMEM((K, 2 * EMBED_DIM), jnp.float32),
        pltpu.VMEM((K, 2 * EMBED_DIM), jnp.float32),
        pltpu.VMEM((K, 2 * EMBED_DIM), jnp.float32),
        pltpu.VMEM((CPW * L,), jnp.float32),
        pltpu.VMEM((CPW * L,), jnp.float32),
        pltpu.VMEM((CPW,), jnp.float32),
        pltpu.VMEM((CPW,), jnp.float32),
        pltpu.SemaphoreType.DMA,
        pltpu.SemaphoreType.DMA,
    ],
)
def _sc_score(*refs):
    _sc_body(*refs)


def kernel(heads, relations, tails, negative_tails, entity_embeddings, relation_embeddings):
    trig = _trig_table(relation_embeddings)
    h = heads.astype(jnp.int32)
    r = relations.astype(jnp.int32)
    t = tails.astype(jnp.int32)
    nt = negative_tails.astype(jnp.int32)
    pos, neg = _sc_score(entity_embeddings, trig, h, r, t, nt)
    return pos, neg


# speedup vs baseline: 1.0272x; 1.0272x over previous
import functools

import jax
import jax.numpy as jnp
from jax import lax
from jax.experimental import pallas as pl
from jax.experimental.pallas import tpu as pltpu
from jax.experimental.pallas import tpu_sc as plsc

NUM_ENTITIES = 1000000
NUM_RELATIONS = 1000
EMBED_DIM = 64
MARGIN = 9.0
EPSILON = 2.0
BATCH = 16384
EMB_RANGE = (MARGIN + EPSILON) / EMBED_DIM
PI = 3.141592653589793

L = 16
NC, NS = 2, 16
NW = NC * NS
CPW = BATCH // NW
K = 64
NCHUNK = CPW // K
NDV = EMBED_DIM // L


def _trig_body(rel_ref, out_ref):
    phase = rel_ref[...] * (PI / EMB_RANGE)
    out_ref[...] = jnp.concatenate([jnp.cos(phase), jnp.sin(phase)], axis=-1)


def _trig_table(rel):
    return pl.pallas_call(
        _trig_body,
        out_shape=jax.ShapeDtypeStruct((NUM_RELATIONS, 2 * EMBED_DIM), jnp.float32),
    )(rel)


def _sqrt16(x):
    i = lax.bitcast_convert_type(x, jnp.int32)
    y = lax.bitcast_convert_type(
        jnp.int32(0x5F1FFFF9) - lax.shift_right_logical(i, 1), jnp.float32)
    y = (0.703952253 * y) * (2.38924456 - x * y * y)
    return x * y


def _sc_body(ent_hbm, trig_hbm, h_hbm, r_hbm, t_hbm, nt_hbm, pos_hbm, neg_hbm,
             hidx, ridx, tidx, ntidx,
             hrow0, trow0, tlrow0, ntrow0,
             hrow1, trow1, tlrow1, ntrow1,
             sump, sumn, posv, negv, sem0, sem1):
    wid = lax.axis_index("s") * NC + lax.axis_index("c")
    base = wid * CPW
    pltpu.sync_copy(h_hbm.at[pl.ds(base, CPW)], hidx)
    pltpu.sync_copy(r_hbm.at[pl.ds(base, CPW)], ridx)
    pltpu.sync_copy(t_hbm.at[pl.ds(base, CPW)], tidx)
    pltpu.sync_copy(nt_hbm.at[pl.ds(base, CPW)], ntidx)

    lane = lax.iota(jnp.int32, L)
    sets = ((hrow0, trow0, tlrow0, ntrow0), (hrow1, trow1, tlrow1, ntrow1))
    sems = (sem0, sem1)

    def issue(c):
        off = c * K
        hrow, trow, tlrow, ntrow = sets[c % 2]
        sem = sems[c % 2]
        return (
            pltpu.async_copy(ent_hbm.at[hidx.at[pl.ds(off, K)]], hrow, sem),
            pltpu.async_copy(trig_hbm.at[ridx.at[pl.ds(off, K)]], trow, sem),
            pltpu.async_copy(ent_hbm.at[tidx.at[pl.ds(off, K)]], tlrow, sem),
            pltpu.async_copy(ent_hbm.at[ntidx.at[pl.ds(off, K)]], ntrow, sem),
        )

    pending = {0: issue(0)}
    for c in range(NCHUNK):
        off = c * K
        hrow, trow, tlrow, ntrow = sets[c % 2]
        if c + 1 < NCHUNK:
            pending[c + 1] = issue(c + 1)
        for cp in pending.pop(c):
            cp.wait()

        @plsc.parallel_loop(0, K, unroll=2)
        def body(i, hrow=hrow, trow=trow, tlrow=tlrow, ntrow=ntrow):
            accp = jnp.zeros((L,), jnp.float32)
            accn = jnp.zeros((L,), jnp.float32)
            for d in range(NDV):
                s = d * L
                re_h = hrow[i, pl.ds(s, L)]
                im_h = hrow[i, pl.ds(EMBED_DIM + s, L)]
                cosr = trow[i, pl.ds(s, L)]
                sinr = trow[i, pl.ds(EMBED_DIM + s, L)]
                rot_re = re_h * cosr - im_h * sinr
                rot_im = re_h * sinr + im_h * cosr
                rep = rot_re - tlrow[i, pl.ds(s, L)]
                imp = rot_im - tlrow[i, pl.ds(EMBED_DIM + s, L)]
                accp = accp + _sqrt16(rep * rep + imp * imp)
                ren = rot_re - ntrow[i, pl.ds(s, L)]
                imn = rot_im - ntrow[i, pl.ds(EMBED_DIM + s, L)]
                accn = accn + _sqrt16(ren * ren + imn * imn)
            sump[pl.ds((off + i) * L, L)] = accp
            sumn[pl.ds((off + i) * L, L)] = accn

    @plsc.parallel_loop(0, CPW // L)
    def red(g):
        rowbase = (g * L + lane) * L
        totp = jnp.zeros((L,), jnp.float32)
        totn = jnp.zeros((L,), jnp.float32)
        for col in range(L):
            idx = rowbase + col
            totp = totp + plsc.load_gather(sump, [idx])
            totn = totn + plsc.load_gather(sumn, [idx])
        posv[pl.ds(g * L, L)] = MARGIN - totp
        negv[pl.ds(g * L, L)] = MARGIN - totn

    pltpu.sync_copy(posv, pos_hbm.at[pl.ds(base, CPW)])
    pltpu.sync_copy(negv, neg_hbm.at[pl.ds(base, CPW)])


@functools.partial(
    pl.kernel,
    out_type=(jax.ShapeDtypeStruct((BATCH,), jnp.float32),
              jax.ShapeDtypeStruct((BATCH,), jnp.float32)),
    mesh=plsc.VectorSubcoreMesh(core_axis_name="c", subcore_axis_name="s"),
    compiler_params=pltpu.CompilerParams(needs_layout_passes=False),
    scratch_types=[
        pltpu.VMEM((CPW,), jnp.int32),
        pltpu.VMEM((CPW,), jnp.int32),
        pltpu.VMEM((CPW,), jnp.int32),
        pltpu.VMEM((CPW,), jnp.int32),
        pltpu.VMEM((K, 2 * EMBED_DIM), jnp.float32),
        pltpu.VMEM((K, 2 * EMBED_DIM), jnp.float32),
        pltpu.VMEM((K, 2 * EMBED_DIM), jnp.float32),
        pltpu.VMEM((K, 2 * EMBED_DIM), jnp.float32),
        pltpu.VMEM((K, 2 * EMBED_DIM), jnp.float32),
        pltpu.VMEM((K, 2 * EMBED_DIM), jnp.float32),
        pltpu.VMEM((K, 2 * EMBED_DIM), jnp.float32),
        pltpu.VMEM((K, 2 * EMBED_DIM), jnp.float32),
        pltpu.VMEM((CPW * L,), jnp.float32),
        pltpu.VMEM((CPW * L,), jnp.float32),
        pltpu.VMEM((CPW,), jnp.float32),
        pltpu.VMEM((CPW,), jnp.float32),
        pltpu.SemaphoreType.DMA,
        pltpu.SemaphoreType.DMA,
    ],
)
def _sc_score(*refs):
    _sc_body(*refs)


def kernel(heads, relations, tails, negative_tails, entity_embeddings, relation_embeddings):
    trig = _trig_table(relation_embeddings)
    h = heads.astype(jnp.int32)
    r = relations.astype(jnp.int32)
    t = tails.astype(jnp.int32)
    nt = negative_tails.astype(jnp.int32)
    pos, neg = _sc_score(entity_embeddings, trig, h, r, t, nt)
    return pos, neg
